# baseline (speedup 1.0000x reference)
import functools

import jax
import jax.numpy as jnp
from jax import lax
from jax.experimental import pallas as pl
from jax.experimental.pallas import tpu as pltpu
from jax.experimental.pallas import tpu_sc as plsc

_BIT_WIDTH = 8
_NC = 2
_NS = 16
_NW = _NC * _NS
_L = 16
_NBINS = 257
_HIST = 528
_CHUNK = 32768


def _sc_histogram(x2d):
    rows, cols = x2d.shape
    rows_per_tile = rows // _NW
    chunk_rows = _CHUNK // cols
    nch = rows_per_tile // chunk_rows
    assert rows_per_tile * _NW == rows and nch * chunk_rows == rows_per_tile
    assert nch % 2 == 0 and cols % (16 * _L) == 0

    mesh = plsc.VectorSubcoreMesh(core_axis_name="c", subcore_axis_name="s")

    @functools.partial(
        pl.kernel,
        out_type=jax.ShapeDtypeStruct((_NW, _HIST), jnp.int32),
        mesh=mesh,
        scratch_types=[
            pltpu.VMEM((chunk_rows, cols), jnp.float32),
            pltpu.VMEM((chunk_rows, cols), jnp.float32),
            pltpu.VMEM((_HIST,), jnp.int32),
            pltpu.SemaphoreType.DMA,
            pltpu.SemaphoreType.DMA,
        ],
        compiler_params=pltpu.CompilerParams(
            needs_layout_passes=False, use_tc_tiling_on_sc=True),
    )
    def hist_kernel(x_hbm, out_hbm, buf0, buf1, hist, sem0, sem1):
        cid = lax.axis_index("c")
        sid = lax.axis_index("s")
        wid = cid * _NS + sid
        base = wid * rows_per_tile

        zeros16 = jnp.zeros((_L,), jnp.int32)

        def _zero(i, carry):
            hist[pl.ds(i * _L, _L)] = zeros16
            return carry

        lax.fori_loop(0, _HIST // _L, _zero, 0)

        ones16 = jnp.ones((_L,), jnp.int32)

        def _start(ci, buf, sem):
            r0 = pl.multiple_of(base + ci * chunk_rows, 8)
            pltpu.async_copy(x_hbm.at[pl.ds(r0, chunk_rows), :], buf, sem)

        def _wait(buf, sem):
            pltpu.make_async_copy(
                x_hbm.at[pl.ds(0, chunk_rows), :], buf, sem).wait()

        def _process(buf):
            cblk = 2

            def _inner(v, carry):
                addrs = []
                for r in range(chunk_rows):
                    for c in range(cblk):
                        xv = buf[r, pl.ds((v * cblk + c) * _L, _L)]
                        bits = lax.bitcast_convert_type(xv, jnp.int32)
                        key = lax.shift_right_logical(
                            bits + jnp.int32(0x007FFFFF), jnp.int32(23))
                        addrs.append(key)
                for a in addrs:
                    plsc.addupdate_scatter(hist, [a], ones16)
                return carry

            lax.fori_loop(0, cols // (_L * cblk), _inner, 0)

        _start(0, buf0, sem0)

        def _outer(p, carry):
            c0 = 2 * p
            _start(c0 + 1, buf1, sem1)
            _wait(buf0, sem0)
            _process(buf0)

            @pl.when(c0 + 2 < nch)
            def _():
                _start(c0 + 2, buf0, sem0)

            _wait(buf1, sem1)
            _process(buf1)
            return carry

        lax.fori_loop(0, nch // 2, _outer, 0)
        pltpu.sync_copy(hist, out_hbm.at[wid])

    return hist_kernel(x2d)


def _tc_quantize(hist2d, x2d, rank):
    rows, cols = x2d.shape
    bm = 512
    grid = rows // bm
    nb2 = 512

    def body(hist_ref, x_ref, o_ref, params):
        @pl.when(pl.program_id(0) == 0)
        def _():
            h = hist_ref[...]
            tot = jnp.sum(h, axis=0, keepdims=True)
            m = tot[:, 0:nb2] + tot[:, 256:256 + nb2]
            tt = jnp.broadcast_to(m, (nb2, nb2))
            bi = lax.broadcasted_iota(jnp.int32, (nb2, nb2), 0)
            ki = lax.broadcasted_iota(jnp.int32, (nb2, nb2), 1)
            tri = jnp.where(ki <= bi, tt, 0)
            cum = jnp.sum(tri, axis=1, keepdims=True)
            bb = jnp.sum((cum < rank).astype(jnp.int32))
            rng = lax.bitcast_convert_type(
                lax.shift_left(bb, 23), jnp.float32)
            inv = lax.bitcast_convert_type(
                lax.shift_left(261 - bb, 23), jnp.float32)
            step = rng * jnp.float32(2.0 ** -(_BIT_WIDTH - 1))
            params[0] = -rng
            params[1] = rng - step
            params[2] = inv
            params[3] = step

        lo = params[0]
        hi = params[1]
        inv = params[2]
        st = params[3]
        xv = x_ref[...]
        c = jnp.minimum(jnp.maximum(xv, lo), hi)
        r = jnp.round(c * inv)
        o_ref[...] = r * st

    return pl.pallas_call(
        body,
        grid=(grid,),
        in_specs=[
            pl.BlockSpec(hist2d.shape, lambda i: (0, 0)),
            pl.BlockSpec((bm, cols), lambda i: (i, 0)),
        ],
        out_specs=pl.BlockSpec((bm, cols), lambda i: (i, 0)),
        out_shape=jax.ShapeDtypeStruct((rows, cols), jnp.float32),
        scratch_shapes=[pltpu.SMEM((4,), jnp.float32)],
    )(hist2d, x2d)


def kernel(input):
    x = input
    n = x.size
    rank = int(0.999 * n) + 1

    cols = 4096
    x2d = x.reshape(n // cols, cols)
    hist_parts = _sc_histogram(x2d)
    hist2d = jnp.pad(hist_parts, ((0, 0), (0, 1024 - _HIST)))

    y2d = _tc_quantize(hist2d, x2d, rank)
    return y2d.reshape(x.shape)


# speedup vs baseline: 1.6498x; 1.6498x over previous
import functools

import jax
import jax.numpy as jnp
from jax import lax
from jax.experimental import pallas as pl
from jax.experimental.pallas import tpu as pltpu
from jax.experimental.pallas import tpu_sc as plsc

_BIT_WIDTH = 8
_NC = 2
_NS = 16
_NW = _NC * _NS
_L = 16
_NBINS = 257
_JUNK = 5
_HIST = 1 << (9 + _JUNK)
_CHUNK = 32768


def _sc_histogram(x2d):
    rows, cols = x2d.shape
    rows_per_tile = rows // _NW
    chunk_rows = _CHUNK // cols
    nch = rows_per_tile // chunk_rows
    assert rows_per_tile * _NW == rows and nch * chunk_rows == rows_per_tile
    assert nch % 2 == 0 and cols % (16 * _L) == 0

    mesh = plsc.VectorSubcoreMesh(core_axis_name="c", subcore_axis_name="s")

    @functools.partial(
        pl.kernel,
        out_type=jax.ShapeDtypeStruct((_NW, _HIST), jnp.int32),
        mesh=mesh,
        scratch_types=[
            pltpu.VMEM((chunk_rows, cols), jnp.float32),
            pltpu.VMEM((chunk_rows, cols), jnp.float32),
            pltpu.VMEM((_HIST,), jnp.int32),
            pltpu.SemaphoreType.DMA,
            pltpu.SemaphoreType.DMA,
        ],
        compiler_params=pltpu.CompilerParams(
            needs_layout_passes=False, use_tc_tiling_on_sc=True),
    )
    def hist_kernel(x_hbm, out_hbm, buf0, buf1, hist, sem0, sem1):
        cid = lax.axis_index("c")
        sid = lax.axis_index("s")
        wid = cid * _NS + sid
        base = wid * rows_per_tile

        zeros16 = jnp.zeros((_L,), jnp.int32)

        def _zero(i, carry):
            hist[pl.ds(i * _L, _L)] = zeros16
            return carry

        lax.fori_loop(0, _HIST // _L, _zero, 0)

        ones16 = jnp.ones((_L,), jnp.int32)

        def _start(ci, buf, sem):
            r0 = pl.multiple_of(base + ci * chunk_rows, 8)
            pltpu.async_copy(x_hbm.at[pl.ds(r0, chunk_rows), :], buf, sem)

        def _wait(buf, sem):
            pltpu.make_async_copy(
                x_hbm.at[pl.ds(0, chunk_rows), :], buf, sem).wait()

        def _process(buf):
            cblk = 2

            def _inner(v, carry):
                addrs = []
                for r in range(chunk_rows):
                    for c in range(cblk):
                        xv = buf[r, pl.ds((v * cblk + c) * _L, _L)]
                        bits = lax.bitcast_convert_type(xv, jnp.int32)
                        addrs.append(lax.shift_right_logical(
                            bits + jnp.int32(0x007FFFFF),
                            jnp.int32(23 - _JUNK)))
                for a in addrs:
                    plsc.addupdate_scatter(hist, [a], ones16)
                return carry

            lax.fori_loop(0, cols // (_L * cblk), _inner, 0)

        _start(0, buf0, sem0)

        def _outer(p, carry):
            c0 = 2 * p
            _start(c0 + 1, buf1, sem1)
            _wait(buf0, sem0)
            _process(buf0)

            @pl.when(c0 + 2 < nch)
            def _():
                _start(c0 + 2, buf0, sem0)

            _wait(buf1, sem1)
            _process(buf1)
            return carry

        lax.fori_loop(0, nch // 2, _outer, 0)
        pltpu.sync_copy(hist, out_hbm.at[wid])

    return hist_kernel(x2d)


def _tc_quantize(hist2d, x2d, rank):
    rows, cols = x2d.shape
    bm = 512
    grid = rows // bm

    def body(hist_ref, x_ref, o_ref, params):
        @pl.when(pl.program_id(0) == 0)
        def _():
            h = hist_ref[...]
            tot = jnp.sum(h, axis=0, keepdims=True)
            half = _HIST // 2
            f = tot[:, 0:half] + tot[:, half:]
            lane = lax.broadcasted_iota(jnp.int32, (1, half), 1)
            p = f
            s = 1
            while s < half:
                shifted = jnp.where(lane < s, 0, pltpu.roll(p, s, 1))
                p = p + shifted
                s *= 2
            bb = lax.shift_right_logical(
                jnp.sum((p < rank).astype(jnp.int32)), _JUNK)
            rng = lax.bitcast_convert_type(
                lax.shift_left(bb, 23), jnp.float32)
            inv = lax.bitcast_convert_type(
                lax.shift_left(261 - bb, 23), jnp.float32)
            step = rng * jnp.float32(2.0 ** -(_BIT_WIDTH - 1))
            params[0] = -rng
            params[1] = rng - step
            params[2] = inv
            params[3] = step

        lo = params[0]
        hi = params[1]
        inv = params[2]
        st = params[3]
        xv = x_ref[...]
        c = jnp.minimum(jnp.maximum(xv, lo), hi)
        r = jnp.round(c * inv)
        o_ref[...] = r * st

    return pl.pallas_call(
        body,
        grid=(grid,),
        in_specs=[
            pl.BlockSpec(hist2d.shape, lambda i: (0, 0)),
            pl.BlockSpec((bm, cols), lambda i: (i, 0)),
        ],
        out_specs=pl.BlockSpec((bm, cols), lambda i: (i, 0)),
        out_shape=jax.ShapeDtypeStruct((rows, cols), jnp.float32),
        scratch_shapes=[pltpu.SMEM((4,), jnp.float32)],
    )(hist2d, x2d)


def kernel(input):
    x = input
    n = x.size
    rank = int(0.999 * n) + 1

    cols = 4096
    x2d = x.reshape(n // cols, cols)
    hist_parts = _sc_histogram(x2d)
    y2d = _tc_quantize(hist_parts, x2d, rank)
    return y2d.reshape(x.shape)
